# Initial kernel scaffold; baseline (speedup 1.0000x reference)
import jax
import jax.numpy as jnp
from jax.experimental import pallas as pl


def kernel(x, W_router, Wg, Wu, Wd, Wg_s, Wu_s, Wd_s):
    raise NotImplementedError("write your pallas kernel here")


# speedup vs baseline: 1.2962x; 1.2962x over previous
import functools

import jax
import jax.numpy as jnp
from jax import lax
from jax.experimental import pallas as pl
from jax.experimental.pallas import tpu as pltpu

S = 2048
DH = 2048
DE = 1024
NE = 8
TBLK = 128
SBLK = 256
NSLOTS = 2 * S + NE * SBLK
NB = NSLOTS // SBLK

_INTERPRET = False


def _silu(v):
    return v * jax.nn.sigmoid(v)


def _shared_router_kernel(x_ref, wg_ref, wu_ref, wd_ref, wr_ref,
                          out_ref, w01_ref, e01_ref):
    xb = x_ref[...]
    g = lax.dot_general(xb, wg_ref[...], (((1,), (1,)), ((), ())),
                        preferred_element_type=jnp.float32)
    u = lax.dot_general(xb, wu_ref[...], (((1,), (1,)), ((), ())),
                        preferred_element_type=jnp.float32)
    gu = _silu(g) * u
    out_ref[...] = lax.dot_general(gu, wd_ref[...], (((1,), (1,)), ((), ())),
                                   preferred_element_type=jnp.float32)

    lg = lax.dot_general(xb, wr_ref[...], (((1,), (1,)), ((), ())),
                         preferred_element_type=jnp.float32)
    m = jnp.max(lg, axis=-1, keepdims=True)
    p = jnp.exp(lg - m)
    sc = p / jnp.sum(p, axis=-1, keepdims=True)
    iota = lax.broadcasted_iota(jnp.int32, (TBLK, NE), 1)
    s0 = jnp.max(sc, axis=-1, keepdims=True)
    a0 = jnp.min(jnp.where(sc == s0, iota, NE), axis=-1, keepdims=True)
    sc1 = jnp.where(iota == a0, -1.0, sc)
    s1 = jnp.max(sc1, axis=-1, keepdims=True)
    a1 = jnp.min(jnp.where(sc1 == s1, iota, NE), axis=-1, keepdims=True)
    w01_ref[...] = jnp.concatenate([s0, s1], axis=1)
    e01_ref[...] = jnp.concatenate([a0, a1], axis=1)


def _expert_kernel(be_ref, x_ref, wg_ref, wu_ref, wd_ref, y_ref):
    del be_ref
    xb = x_ref[...]
    g = lax.dot_general(xb, wg_ref[0], (((1,), (1,)), ((), ())),
                        preferred_element_type=jnp.float32)
    u = lax.dot_general(xb, wu_ref[0], (((1,), (1,)), ((), ())),
                        preferred_element_type=jnp.float32)
    h = _silu(g) * u
    y_ref[...] = lax.dot_general(h, wd_ref[0], (((1,), (1,)), ((), ())),
                                 preferred_element_type=jnp.float32)


def _routing_metadata(e01, w01):
    e_all = jnp.concatenate([e01[:, 0], e01[:, 1]])
    w_all = jnp.concatenate([w01[:, 0], w01[:, 1]])
    t_all = jnp.concatenate([jnp.arange(S, dtype=jnp.int32)] * 2)
    onehot = (e_all[:, None] == jnp.arange(NE, dtype=jnp.int32)[None, :])
    oh_i = onehot.astype(jnp.int32)
    counts = jnp.sum(oh_i, axis=0)
    padded = ((counts + SBLK - 1) // SBLK) * SBLK
    start = jnp.concatenate([jnp.zeros((1,), jnp.int32),
                             jnp.cumsum(padded)[:-1].astype(jnp.int32)])
    rank = jnp.sum((jnp.cumsum(oh_i, axis=0) - oh_i) * oh_i, axis=1)
    pos = start[e_all] + rank
    slot_token = jnp.zeros((NSLOTS,), jnp.int32).at[pos].set(t_all)
    blk_off = jnp.arange(NB, dtype=jnp.int32) * SBLK
    block_expert = jnp.sum(
        (blk_off[:, None] >= start[None, 1:]).astype(jnp.int32), axis=1)
    return slot_token, block_expert, pos[:S], pos[S:], w_all


def kernel(x, W_router, Wg, Wu, Wd, Wg_s, Wu_s, Wd_s):
    x_flat = x.reshape(S, DH)

    shared_out, w01, e01 = pl.pallas_call(
        _shared_router_kernel,
        grid=(S // TBLK,),
        in_specs=[
            pl.BlockSpec((TBLK, DH), lambda b: (b, 0)),
            pl.BlockSpec((2 * DE, DH), lambda b: (0, 0)),
            pl.BlockSpec((2 * DE, DH), lambda b: (0, 0)),
            pl.BlockSpec((DH, 2 * DE), lambda b: (0, 0)),
            pl.BlockSpec((NE, DH), lambda b: (0, 0)),
        ],
        out_specs=[
            pl.BlockSpec((TBLK, DH), lambda b: (b, 0)),
            pl.BlockSpec((TBLK, 2), lambda b: (b, 0)),
            pl.BlockSpec((TBLK, 2), lambda b: (b, 0)),
        ],
        out_shape=[
            jax.ShapeDtypeStruct((S, DH), jnp.float32),
            jax.ShapeDtypeStruct((S, 2), jnp.float32),
            jax.ShapeDtypeStruct((S, 2), jnp.int32),
        ],
        interpret=_INTERPRET,
    )(x_flat, Wg_s, Wu_s, Wd_s, W_router)

    slot_token, block_expert, pos0, pos1, _ = _routing_metadata(e01, w01)

    x_sorted = x_flat[slot_token]

    y_slots = pl.pallas_call(
        _expert_kernel,
        grid_spec=pltpu.PrefetchScalarGridSpec(
            num_scalar_prefetch=1,
            grid=(NB,),
            in_specs=[
                pl.BlockSpec((SBLK, DH), lambda b, be: (b, 0)),
                pl.BlockSpec((1, DE, DH), lambda b, be: (be[b], 0, 0)),
                pl.BlockSpec((1, DE, DH), lambda b, be: (be[b], 0, 0)),
                pl.BlockSpec((1, DH, DE), lambda b, be: (be[b], 0, 0)),
            ],
            out_specs=pl.BlockSpec((SBLK, DH), lambda b, be: (b, 0)),
        ),
        out_shape=jax.ShapeDtypeStruct((NSLOTS, DH), jnp.float32),
        interpret=_INTERPRET,
    )(block_expert, x_sorted, Wg, Wu, Wd)

    routed = (y_slots[pos0] * w01[:, 0:1] + y_slots[pos1] * w01[:, 1:2])
    return (routed + shared_out).reshape(x.shape)
